# baseline (speedup 1.0000x reference)
import jax
import jax.numpy as jnp
from jax import lax
from jax.experimental import pallas as pl
from jax.experimental.pallas import tpu as pltpu

FUSED_VOCAB = 128
EDGE_TILE = 8192
NODE_PACK = 4
EDGE_PACK = 32
EDGE_SUB = 8


def _round_up(x, m):
    return ((x + m - 1) // m) * m


def kernel(emb_w, tag_w, per_w, grp_w, lin_w, lin_b, lin_e_w, lin_e_b,
           period_table, group_table, z, tag, rel_pos, edge_attr):
    n = z.shape[0]
    e = rel_pos.shape[0]
    n_elements = emb_w.shape[0]
    atom_dim = emb_w.shape[1]
    tag_dim = tag_w.shape[1]
    pg_dim = per_w.shape[1]
    hidden = lin_w.shape[1]
    rp_dim = rel_pos.shape[1]
    ea_dim = edge_attr.shape[1]

    def fused_kernel(z4_ref, t4_ref, c_ref, rp_ref, ea_ref,
                     wr_ref, we_ref, be_ref, h4_ref, e32_ref):
        rows = z4_ref.shape[0]
        lanes = lax.broadcasted_iota(jnp.int32, (rows, FUSED_VOCAB), 1)
        pieces = []
        for k in range(NODE_PACK):
            mh = ((lanes == z4_ref[:, k:k + 1])
                  | (lanes == t4_ref[:, k:k + 1] + n_elements))
            pieces.append(jnp.dot(mh.astype(jnp.float32), c_ref[...],
                                  preferred_element_type=jnp.float32))
        h4_ref[...] = jnp.concatenate(pieces, axis=1)

        ea_blk = ea_ref[...]
        w = EDGE_SUB * ea_dim
        parts = [jnp.dot(ea_blk[:, w * k:w * (k + 1)], we_ref[...],
                         preferred_element_type=jnp.float32)
                 for k in range(EDGE_PACK // EDGE_SUB)]
        e32_ref[...] = (jnp.concatenate(parts, axis=1)
                        + jnp.dot(rp_ref[...], wr_ref[...],
                                  preferred_element_type=jnp.float32)
                        + be_ref[...])

    emb_eff = jnp.dot(emb_w, lin_w[:atom_dim], preferred_element_type=jnp.float32)
    tag_eff = jnp.dot(tag_w, lin_w[atom_dim:atom_dim + tag_dim],
                      preferred_element_type=jnp.float32)
    per_eff = jnp.dot(per_w, lin_w[atom_dim + tag_dim:atom_dim + tag_dim + pg_dim],
                      preferred_element_type=jnp.float32)
    grp_eff = jnp.dot(grp_w, lin_w[atom_dim + tag_dim + pg_dim:],
                      preferred_element_type=jnp.float32)
    a_rows = (emb_eff + per_eff[period_table] + grp_eff[group_table]
              + lin_b.astype(jnp.float32))
    c = jnp.zeros((FUSED_VOCAB, hidden), jnp.float32)
    c = lax.dynamic_update_slice(c, a_rows, (0, 0))
    c = lax.dynamic_update_slice(c, tag_eff, (n_elements, 0))

    rp_dim4 = _round_up(rp_dim, 4)
    w3 = jnp.pad(lin_e_w[:rp_dim].astype(jnp.float32),
                 ((0, rp_dim4 - rp_dim), (0, 0)))
    w16 = lin_e_w[rp_dim:].astype(jnp.float32)
    wr = jnp.kron(jnp.eye(EDGE_PACK, dtype=jnp.float32), w3)
    we = jnp.kron(jnp.eye(EDGE_SUB, dtype=jnp.float32), w16)
    b32 = jnp.tile(lin_e_b.astype(jnp.float32), (1, EDGE_PACK))

    e_pad = _round_up(max(e, 1), EDGE_TILE)
    g = e_pad // EDGE_TILE
    tn = _round_up(-(-max(n, 1) // g), 8 * NODE_PACK)
    n_pad = g * tn
    zc = z.astype(jnp.int32)
    tc = tag.astype(jnp.int32)
    rp = rel_pos.astype(jnp.float32)
    ea = edge_attr.astype(jnp.float32)
    if n_pad != n:
        zc = jnp.pad(zc, (0, n_pad - n))
        tc = jnp.pad(tc, (0, n_pad - n))
    if e_pad != e:
        rp = jnp.pad(rp, ((0, e_pad - e), (0, 0)))
        ea = jnp.pad(ea, ((0, e_pad - e), (0, 0)))

    z4 = zc.reshape(n_pad // NODE_PACK, NODE_PACK)
    t4 = tc.reshape(n_pad // NODE_PACK, NODE_PACK)
    rp32 = jnp.pad(rp, ((0, 0), (0, rp_dim4 - rp_dim))).reshape(
        e_pad // EDGE_PACK, EDGE_PACK * rp_dim4)
    ea32 = ea.reshape(e_pad // EDGE_PACK, EDGE_PACK * ea_dim)

    tn4 = tn // NODE_PACK
    te32 = EDGE_TILE // EDGE_PACK

    h4, e32 = pl.pallas_call(
        fused_kernel,
        out_shape=(jax.ShapeDtypeStruct((n_pad // NODE_PACK, NODE_PACK * hidden),
                                        jnp.float32),
                   jax.ShapeDtypeStruct((e_pad // EDGE_PACK, EDGE_PACK * hidden),
                                        jnp.float32)),
        grid=(g,),
        in_specs=[
            pl.BlockSpec((tn4, NODE_PACK), lambda i: (i, 0)),
            pl.BlockSpec((tn4, NODE_PACK), lambda i: (i, 0)),
            pl.BlockSpec((FUSED_VOCAB, hidden), lambda i: (0, 0)),
            pl.BlockSpec((te32, EDGE_PACK * rp_dim4), lambda i: (i, 0)),
            pl.BlockSpec((te32, EDGE_PACK * ea_dim), lambda i: (i, 0)),
            pl.BlockSpec((EDGE_PACK * rp_dim4, EDGE_PACK * hidden),
                         lambda i: (0, 0)),
            pl.BlockSpec((EDGE_SUB * ea_dim, EDGE_SUB * hidden),
                         lambda i: (0, 0)),
            pl.BlockSpec((1, EDGE_PACK * hidden), lambda i: (0, 0)),
        ],
        out_specs=(pl.BlockSpec((tn4, NODE_PACK * hidden), lambda i: (i, 0)),
                   pl.BlockSpec((te32, EDGE_PACK * hidden), lambda i: (i, 0))),
        compiler_params=pltpu.CompilerParams(
            dimension_semantics=("parallel",)),
    )(z4, t4, c, rp32, ea32, wr, we, b32)

    h = h4.reshape(n_pad, hidden)
    e_out = e32.reshape(e_pad, hidden)
    if n_pad != n:
        h = h[:n]
    if e_pad != e:
        e_out = e_out[:e]
    return h, e_out


# speedup vs baseline: 17.9911x; 17.9911x over previous
import jax
import jax.numpy as jnp
from jax import lax
from jax.experimental import pallas as pl
from jax.experimental.pallas import tpu as pltpu

FUSED_VOCAB = 128
EDGE_TILE = 8192


def _round_up(x, m):
    return ((x + m - 1) // m) * m


def kernel(emb_w, tag_w, per_w, grp_w, lin_w, lin_b, lin_e_w, lin_e_b,
           period_table, group_table, z, tag, rel_pos, edge_attr):
    n = z.shape[0]
    e = rel_pos.shape[0]
    n_elements = emb_w.shape[0]
    atom_dim = emb_w.shape[1]
    tag_dim = tag_w.shape[1]
    pg_dim = per_w.shape[1]
    hidden = lin_w.shape[1]
    rp_dim = rel_pos.shape[1]
    ea_dim = edge_attr.shape[1]

    def fused_kernel(z_ref, t_ref, ct_ref, rp_ref, ea_ref,
                     w3t_ref, w16t_ref, bt_ref, ht_ref, et_ref):
        cols = z_ref.shape[0]
        vocab = lax.broadcasted_iota(jnp.int32, (FUSED_VOCAB, cols), 0)
        zrow = jnp.broadcast_to(z_ref[...][None, :], (FUSED_VOCAB, cols))
        trow = jnp.broadcast_to(t_ref[...][None, :] + n_elements,
                                (FUSED_VOCAB, cols))
        mh = ((vocab == zrow) | (vocab == trow)).astype(jnp.float32)
        ht_ref[...] = jnp.dot(ct_ref[...], mh,
                              preferred_element_type=jnp.float32)
        et_ref[...] = (jnp.dot(w3t_ref[...], rp_ref[...],
                               preferred_element_type=jnp.float32)
                       + jnp.dot(w16t_ref[...], ea_ref[...],
                                 preferred_element_type=jnp.float32)
                       + bt_ref[...])

    emb_eff = jnp.dot(emb_w, lin_w[:atom_dim], preferred_element_type=jnp.float32)
    tag_eff = jnp.dot(tag_w, lin_w[atom_dim:atom_dim + tag_dim],
                      preferred_element_type=jnp.float32)
    per_eff = jnp.dot(per_w, lin_w[atom_dim + tag_dim:atom_dim + tag_dim + pg_dim],
                      preferred_element_type=jnp.float32)
    grp_eff = jnp.dot(grp_w, lin_w[atom_dim + tag_dim + pg_dim:],
                      preferred_element_type=jnp.float32)
    a_rows = (emb_eff + per_eff[period_table] + grp_eff[group_table]
              + lin_b.astype(jnp.float32))
    c = jnp.zeros((FUSED_VOCAB, hidden), jnp.float32)
    c = lax.dynamic_update_slice(c, a_rows, (0, 0))
    c = lax.dynamic_update_slice(c, tag_eff, (n_elements, 0))
    ct = c.T

    w3t = lin_e_w[:rp_dim].astype(jnp.float32).T
    w16t = lin_e_w[rp_dim:].astype(jnp.float32).T
    bt = lin_e_b.astype(jnp.float32).T

    rp_t = rel_pos.astype(jnp.float32).T
    ea_t = edge_attr.astype(jnp.float32).T
    zc = z.astype(jnp.int32)
    tc = tag.astype(jnp.int32)

    e_pad = _round_up(max(e, 1), EDGE_TILE)
    g = e_pad // EDGE_TILE
    tn = _round_up(-(-max(n, 1) // g), 128)
    n_pad = g * tn
    if n_pad != n:
        zc = jnp.pad(zc, (0, n_pad - n))
        tc = jnp.pad(tc, (0, n_pad - n))
    if e_pad != e:
        rp_t = jnp.pad(rp_t, ((0, 0), (0, e_pad - e)))
        ea_t = jnp.pad(ea_t, ((0, 0), (0, e_pad - e)))

    ht, et = pl.pallas_call(
        fused_kernel,
        out_shape=(jax.ShapeDtypeStruct((hidden, n_pad), jnp.float32),
                   jax.ShapeDtypeStruct((hidden, e_pad), jnp.float32)),
        grid=(g,),
        in_specs=[
            pl.BlockSpec((tn,), lambda i: (i,)),
            pl.BlockSpec((tn,), lambda i: (i,)),
            pl.BlockSpec((hidden, FUSED_VOCAB), lambda i: (0, 0)),
            pl.BlockSpec((rp_dim, EDGE_TILE), lambda i: (0, i)),
            pl.BlockSpec((ea_dim, EDGE_TILE), lambda i: (0, i)),
            pl.BlockSpec((hidden, rp_dim), lambda i: (0, 0)),
            pl.BlockSpec((hidden, ea_dim), lambda i: (0, 0)),
            pl.BlockSpec((hidden, 1), lambda i: (0, 0)),
        ],
        out_specs=(pl.BlockSpec((hidden, tn), lambda i: (0, i)),
                   pl.BlockSpec((hidden, EDGE_TILE), lambda i: (0, i))),
        compiler_params=pltpu.CompilerParams(
            dimension_semantics=("parallel",)),
    )(zc, tc, ct, rp_t, ea_t, w3t, w16t, bt)

    h = ht.T if n_pad == n else ht[:, :n].T
    e_out = et.T if e_pad == e else et[:, :e].T
    return h, e_out


# speedup vs baseline: 25.0746x; 1.3937x over previous
import jax
import jax.numpy as jnp
from jax import lax
from jax.experimental import pallas as pl
from jax.experimental.pallas import tpu as pltpu

FUSED_VOCAB = 128
EDGE_TILE = 16384


def _round_up(x, m):
    return ((x + m - 1) // m) * m


def kernel(emb_w, tag_w, per_w, grp_w, lin_w, lin_b, lin_e_w, lin_e_b,
           period_table, group_table, z, tag, rel_pos, edge_attr):
    n = z.shape[0]
    e = rel_pos.shape[0]
    n_elements = emb_w.shape[0]
    atom_dim = emb_w.shape[1]
    tag_dim = tag_w.shape[1]
    pg_dim = per_w.shape[1]
    hidden = lin_w.shape[1]
    rp_dim = rel_pos.shape[1]
    ea_dim = edge_attr.shape[1]

    def fused_kernel(z_ref, t_ref, wp_ref, rp_ref, ea_ref, ht_ref, et_ref):
        ct = wp_ref[:, 0:FUSED_VOCAB]
        w3t = wp_ref[:, FUSED_VOCAB:FUSED_VOCAB + rp_dim]
        w16t = wp_ref[:, 2 * FUSED_VOCAB:2 * FUSED_VOCAB + ea_dim]
        bt = wp_ref[:, 3 * FUSED_VOCAB:3 * FUSED_VOCAB + 1]
        cols = z_ref.shape[0]
        vocab = lax.broadcasted_iota(jnp.int32, (FUSED_VOCAB, cols), 0)
        zrow = jnp.broadcast_to(z_ref[...][None, :], (FUSED_VOCAB, cols))
        trow = jnp.broadcast_to(t_ref[...][None, :] + n_elements,
                                (FUSED_VOCAB, cols))
        mh = ((vocab == zrow) | (vocab == trow)).astype(jnp.float32)
        ht_ref[...] = jnp.dot(ct, mh, preferred_element_type=jnp.float32)
        et_ref[...] = (jnp.dot(w3t, rp_ref[...],
                               preferred_element_type=jnp.float32)
                       + jnp.dot(w16t, ea_ref[...],
                                 preferred_element_type=jnp.float32)
                       + bt)

    emb_eff = jnp.dot(emb_w, lin_w[:atom_dim], preferred_element_type=jnp.float32)
    tag_eff = jnp.dot(tag_w, lin_w[atom_dim:atom_dim + tag_dim],
                      preferred_element_type=jnp.float32)
    per_eff = jnp.dot(per_w, lin_w[atom_dim + tag_dim:atom_dim + tag_dim + pg_dim],
                      preferred_element_type=jnp.float32)
    grp_eff = jnp.dot(grp_w, lin_w[atom_dim + tag_dim + pg_dim:],
                      preferred_element_type=jnp.float32)
    a_rows = (emb_eff + per_eff[period_table] + grp_eff[group_table]
              + lin_b.astype(jnp.float32))
    c = jnp.zeros((FUSED_VOCAB, hidden), jnp.float32)
    c = lax.dynamic_update_slice(c, a_rows, (0, 0))
    c = lax.dynamic_update_slice(c, tag_eff, (n_elements, 0))
    wpack = jnp.zeros((hidden, 4 * FUSED_VOCAB), jnp.float32)
    wpack = lax.dynamic_update_slice(wpack, c.T, (0, 0))
    wpack = lax.dynamic_update_slice(
        wpack, lin_e_w[:rp_dim].astype(jnp.float32).T, (0, FUSED_VOCAB))
    wpack = lax.dynamic_update_slice(
        wpack, lin_e_w[rp_dim:].astype(jnp.float32).T, (0, 2 * FUSED_VOCAB))
    wpack = lax.dynamic_update_slice(
        wpack, lin_e_b.astype(jnp.float32).T, (0, 3 * FUSED_VOCAB))

    rp_t = rel_pos.astype(jnp.float32).T
    ea_t = edge_attr.astype(jnp.float32).T
    zc = z.astype(jnp.int32)
    tc = tag.astype(jnp.int32)

    e_pad = _round_up(max(e, 1), EDGE_TILE)
    g = e_pad // EDGE_TILE
    tn = _round_up(-(-max(n, 1) // g), 128)
    n_pad = g * tn
    if n_pad != n:
        zc = jnp.pad(zc, (0, n_pad - n))
        tc = jnp.pad(tc, (0, n_pad - n))
    if e_pad != e:
        rp_t = jnp.pad(rp_t, ((0, 0), (0, e_pad - e)))
        ea_t = jnp.pad(ea_t, ((0, 0), (0, e_pad - e)))

    ht, et = pl.pallas_call(
        fused_kernel,
        out_shape=(jax.ShapeDtypeStruct((hidden, n_pad), jnp.float32),
                   jax.ShapeDtypeStruct((hidden, e_pad), jnp.float32)),
        grid=(g,),
        in_specs=[
            pl.BlockSpec((tn,), lambda i: (i,)),
            pl.BlockSpec((tn,), lambda i: (i,)),
            pl.BlockSpec((hidden, 4 * FUSED_VOCAB), lambda i: (0, 0)),
            pl.BlockSpec((rp_dim, EDGE_TILE), lambda i: (0, i)),
            pl.BlockSpec((ea_dim, EDGE_TILE), lambda i: (0, i)),
        ],
        out_specs=(pl.BlockSpec((hidden, tn), lambda i: (0, i)),
                   pl.BlockSpec((hidden, EDGE_TILE), lambda i: (0, i))),
        compiler_params=pltpu.CompilerParams(
            dimension_semantics=("parallel",)),
    )(zc, tc, wpack, rp_t, ea_t)

    h = ht.T if n_pad == n else ht[:, :n].T
    e_out = et.T if e_pad == e else et[:, :e].T
    return h, e_out


# speedup vs baseline: 28.5247x; 1.1376x over previous
import jax
import jax.numpy as jnp
from jax import lax
from jax.experimental import pallas as pl
from jax.experimental.pallas import tpu as pltpu

FUSED_VOCAB = 128
EDGE_TILE = 32768


def _round_up(x, m):
    return ((x + m - 1) // m) * m


def kernel(emb_w, tag_w, per_w, grp_w, lin_w, lin_b, lin_e_w, lin_e_b,
           period_table, group_table, z, tag, rel_pos, edge_attr):
    n = z.shape[0]
    e = rel_pos.shape[0]
    n_elements = emb_w.shape[0]
    atom_dim = emb_w.shape[1]
    tag_dim = tag_w.shape[1]
    pg_dim = per_w.shape[1]
    hidden = lin_w.shape[1]
    rp_dim = rel_pos.shape[1]
    ea_dim = edge_attr.shape[1]

    def fused_kernel(z_ref, t_ref, wp_ref, rp_ref, ea_ref, ht_ref, et_ref):
        ct = wp_ref[:, 0:FUSED_VOCAB]
        w3t = wp_ref[:, FUSED_VOCAB:FUSED_VOCAB + rp_dim]
        w16t = wp_ref[:, 2 * FUSED_VOCAB:2 * FUSED_VOCAB + ea_dim]
        bt = wp_ref[:, 3 * FUSED_VOCAB:3 * FUSED_VOCAB + 1]
        cols = z_ref.shape[0]
        vocab = lax.broadcasted_iota(jnp.int32, (FUSED_VOCAB, cols), 0)
        zrow = jnp.broadcast_to(z_ref[...][None, :], (FUSED_VOCAB, cols))
        trow = jnp.broadcast_to(t_ref[...][None, :] + n_elements,
                                (FUSED_VOCAB, cols))
        mh = ((vocab == zrow) | (vocab == trow)).astype(jnp.float32)
        ht_ref[...] = jnp.dot(ct, mh, preferred_element_type=jnp.float32)
        et_ref[...] = (jnp.dot(w3t, rp_ref[...],
                               preferred_element_type=jnp.float32)
                       + jnp.dot(w16t, ea_ref[...],
                                 preferred_element_type=jnp.float32)
                       + bt)

    emb_eff = jnp.dot(emb_w, lin_w[:atom_dim], preferred_element_type=jnp.float32)
    tag_eff = jnp.dot(tag_w, lin_w[atom_dim:atom_dim + tag_dim],
                      preferred_element_type=jnp.float32)
    per_eff = jnp.dot(per_w, lin_w[atom_dim + tag_dim:atom_dim + tag_dim + pg_dim],
                      preferred_element_type=jnp.float32)
    grp_eff = jnp.dot(grp_w, lin_w[atom_dim + tag_dim + pg_dim:],
                      preferred_element_type=jnp.float32)
    a_rows = (emb_eff + per_eff[period_table] + grp_eff[group_table]
              + lin_b.astype(jnp.float32))
    c = jnp.zeros((FUSED_VOCAB, hidden), jnp.float32)
    c = lax.dynamic_update_slice(c, a_rows, (0, 0))
    c = lax.dynamic_update_slice(c, tag_eff, (n_elements, 0))
    wpack = jnp.zeros((hidden, 4 * FUSED_VOCAB), jnp.float32)
    wpack = lax.dynamic_update_slice(wpack, c.T, (0, 0))
    wpack = lax.dynamic_update_slice(
        wpack, lin_e_w[:rp_dim].astype(jnp.float32).T, (0, FUSED_VOCAB))
    wpack = lax.dynamic_update_slice(
        wpack, lin_e_w[rp_dim:].astype(jnp.float32).T, (0, 2 * FUSED_VOCAB))
    wpack = lax.dynamic_update_slice(
        wpack, lin_e_b.astype(jnp.float32).T, (0, 3 * FUSED_VOCAB))

    rp_t = rel_pos.astype(jnp.float32).T
    ea_t = edge_attr.astype(jnp.float32).T
    zc = z.astype(jnp.int32)
    tc = tag.astype(jnp.int32)

    e_pad = _round_up(max(e, 1), EDGE_TILE)
    g = e_pad // EDGE_TILE
    tn = _round_up(-(-max(n, 1) // g), 128)
    n_pad = g * tn
    if n_pad != n:
        zc = jnp.pad(zc, (0, n_pad - n))
        tc = jnp.pad(tc, (0, n_pad - n))
    if e_pad != e:
        rp_t = jnp.pad(rp_t, ((0, 0), (0, e_pad - e)))
        ea_t = jnp.pad(ea_t, ((0, 0), (0, e_pad - e)))

    ht, et = pl.pallas_call(
        fused_kernel,
        out_shape=(jax.ShapeDtypeStruct((hidden, n_pad), jnp.float32),
                   jax.ShapeDtypeStruct((hidden, e_pad), jnp.float32)),
        grid=(g,),
        in_specs=[
            pl.BlockSpec((tn,), lambda i: (i,)),
            pl.BlockSpec((tn,), lambda i: (i,)),
            pl.BlockSpec((hidden, 4 * FUSED_VOCAB), lambda i: (0, 0)),
            pl.BlockSpec((rp_dim, EDGE_TILE), lambda i: (0, i)),
            pl.BlockSpec((ea_dim, EDGE_TILE), lambda i: (0, i)),
        ],
        out_specs=(pl.BlockSpec((hidden, tn), lambda i: (0, i)),
                   pl.BlockSpec((hidden, EDGE_TILE), lambda i: (0, i))),
        compiler_params=pltpu.CompilerParams(
            dimension_semantics=("parallel",)),
    )(zc, tc, wpack, rp_t, ea_t)

    h = ht.T if n_pad == n else ht[:, :n].T
    e_out = et.T if e_pad == e else et[:, :e].T
    return h, e_out


# speedup vs baseline: 28.7549x; 1.0081x over previous
import jax
import jax.numpy as jnp
from jax import lax
from jax.experimental import pallas as pl
from jax.experimental.pallas import tpu as pltpu

FUSED_VOCAB = 128
EDGE_TILE = 65536


def _round_up(x, m):
    return ((x + m - 1) // m) * m


def kernel(emb_w, tag_w, per_w, grp_w, lin_w, lin_b, lin_e_w, lin_e_b,
           period_table, group_table, z, tag, rel_pos, edge_attr):
    n = z.shape[0]
    e = rel_pos.shape[0]
    n_elements = emb_w.shape[0]
    atom_dim = emb_w.shape[1]
    tag_dim = tag_w.shape[1]
    pg_dim = per_w.shape[1]
    hidden = lin_w.shape[1]
    rp_dim = rel_pos.shape[1]
    ea_dim = edge_attr.shape[1]

    def fused_kernel(z_ref, t_ref, wp_ref, rp_ref, ea_ref, ht_ref, et_ref):
        ct = wp_ref[:, 0:FUSED_VOCAB]
        w3t = wp_ref[:, FUSED_VOCAB:FUSED_VOCAB + rp_dim]
        w16t = wp_ref[:, 2 * FUSED_VOCAB:2 * FUSED_VOCAB + ea_dim]
        bt = wp_ref[:, 3 * FUSED_VOCAB:3 * FUSED_VOCAB + 1]
        cols = z_ref.shape[0]
        vocab = lax.broadcasted_iota(jnp.int32, (FUSED_VOCAB, cols), 0)
        zrow = jnp.broadcast_to(z_ref[...][None, :], (FUSED_VOCAB, cols))
        trow = jnp.broadcast_to(t_ref[...][None, :] + n_elements,
                                (FUSED_VOCAB, cols))
        mh = ((vocab == zrow) | (vocab == trow)).astype(jnp.float32)
        ht_ref[...] = jnp.dot(ct, mh, preferred_element_type=jnp.float32)
        et_ref[...] = (jnp.dot(w3t, rp_ref[...],
                               preferred_element_type=jnp.float32)
                       + jnp.dot(w16t, ea_ref[...],
                                 preferred_element_type=jnp.float32)
                       + bt)

    emb_eff = jnp.dot(emb_w, lin_w[:atom_dim], preferred_element_type=jnp.float32)
    tag_eff = jnp.dot(tag_w, lin_w[atom_dim:atom_dim + tag_dim],
                      preferred_element_type=jnp.float32)
    per_eff = jnp.dot(per_w, lin_w[atom_dim + tag_dim:atom_dim + tag_dim + pg_dim],
                      preferred_element_type=jnp.float32)
    grp_eff = jnp.dot(grp_w, lin_w[atom_dim + tag_dim + pg_dim:],
                      preferred_element_type=jnp.float32)
    a_rows = (emb_eff + per_eff[period_table] + grp_eff[group_table]
              + lin_b.astype(jnp.float32))
    c = jnp.zeros((FUSED_VOCAB, hidden), jnp.float32)
    c = lax.dynamic_update_slice(c, a_rows, (0, 0))
    c = lax.dynamic_update_slice(c, tag_eff, (n_elements, 0))
    wpack = jnp.zeros((hidden, 4 * FUSED_VOCAB), jnp.float32)
    wpack = lax.dynamic_update_slice(wpack, c.T, (0, 0))
    wpack = lax.dynamic_update_slice(
        wpack, lin_e_w[:rp_dim].astype(jnp.float32).T, (0, FUSED_VOCAB))
    wpack = lax.dynamic_update_slice(
        wpack, lin_e_w[rp_dim:].astype(jnp.float32).T, (0, 2 * FUSED_VOCAB))
    wpack = lax.dynamic_update_slice(
        wpack, lin_e_b.astype(jnp.float32).T, (0, 3 * FUSED_VOCAB))

    rp_t = rel_pos.astype(jnp.float32).T
    ea_t = edge_attr.astype(jnp.float32).T
    zc = z.astype(jnp.int32)
    tc = tag.astype(jnp.int32)

    e_pad = _round_up(max(e, 1), EDGE_TILE)
    g = e_pad // EDGE_TILE
    tn = _round_up(-(-max(n, 1) // g), 128)
    n_pad = g * tn
    if n_pad != n:
        zc = jnp.pad(zc, (0, n_pad - n))
        tc = jnp.pad(tc, (0, n_pad - n))
    if e_pad != e:
        rp_t = jnp.pad(rp_t, ((0, 0), (0, e_pad - e)))
        ea_t = jnp.pad(ea_t, ((0, 0), (0, e_pad - e)))

    ht, et = pl.pallas_call(
        fused_kernel,
        out_shape=(jax.ShapeDtypeStruct((hidden, n_pad), jnp.float32),
                   jax.ShapeDtypeStruct((hidden, e_pad), jnp.float32)),
        grid=(g,),
        in_specs=[
            pl.BlockSpec((tn,), lambda i: (i,)),
            pl.BlockSpec((tn,), lambda i: (i,)),
            pl.BlockSpec((hidden, 4 * FUSED_VOCAB), lambda i: (0, 0)),
            pl.BlockSpec((rp_dim, EDGE_TILE), lambda i: (0, i)),
            pl.BlockSpec((ea_dim, EDGE_TILE), lambda i: (0, i)),
        ],
        out_specs=(pl.BlockSpec((hidden, tn), lambda i: (0, i)),
                   pl.BlockSpec((hidden, EDGE_TILE), lambda i: (0, i))),
        compiler_params=pltpu.CompilerParams(
            dimension_semantics=("parallel",)),
    )(zc, tc, wpack, rp_t, ea_t)

    h = ht.T if n_pad == n else ht[:, :n].T
    e_out = et.T if e_pad == e else et[:, :e].T
    return h, e_out


# speedup vs baseline: 29.7820x; 1.0357x over previous
import jax
import jax.numpy as jnp
from jax import lax
from jax.experimental import pallas as pl
from jax.experimental.pallas import tpu as pltpu

EDGE_TILE = 65536


def _round_up(x, m):
    return ((x + m - 1) // m) * m


def kernel(emb_w, tag_w, per_w, grp_w, lin_w, lin_b, lin_e_w, lin_e_b,
           period_table, group_table, z, tag, rel_pos, edge_attr):
    n = z.shape[0]
    e = rel_pos.shape[0]
    n_elements = emb_w.shape[0]
    n_tags = tag_w.shape[0]
    atom_dim = emb_w.shape[1]
    tag_dim = tag_w.shape[1]
    pg_dim = per_w.shape[1]
    n_periods = per_w.shape[0]
    n_groups = grp_w.shape[0]
    hidden = lin_w.shape[1]
    rp_dim = rel_pos.shape[1]
    ea_dim = edge_attr.shape[1]
    vocab = n_elements + n_tags

    def fused_kernel(z_ref, t_ref, pts_ref, embt_ref, tagt_ref, pert_ref,
                     grpt_ref, lwt_ref, lbt_ref, lewt_ref, lebt_ref,
                     rp_ref, ea_ref, ht_ref, et_ref):
        f32 = jnp.float32
        lwt = lwt_ref[...]
        emb_efft = jnp.dot(lwt[:, :atom_dim], embt_ref[...],
                           preferred_element_type=f32)
        tag_efft = jnp.dot(lwt[:, atom_dim:atom_dim + tag_dim], tagt_ref[...],
                           preferred_element_type=f32)
        per_efft = jnp.dot(
            lwt[:, atom_dim + tag_dim:atom_dim + tag_dim + pg_dim],
            pert_ref[...], preferred_element_type=f32)
        grp_efft = jnp.dot(lwt[:, atom_dim + tag_dim + pg_dim:], grpt_ref[...],
                           preferred_element_type=f32)
        pts = pts_ref[...]
        pmask = (lax.broadcasted_iota(jnp.int32, (n_periods, n_elements), 0)
                 == pts[0:1, :]).astype(f32)
        gmask = (lax.broadcasted_iota(jnp.int32, (n_groups, n_elements), 0)
                 == pts[1:2, :]).astype(f32)
        at = (emb_efft
              + jnp.dot(per_efft, pmask, preferred_element_type=f32)
              + jnp.dot(grp_efft, gmask, preferred_element_type=f32)
              + lbt_ref[...])
        ct = jnp.concatenate([at, tag_efft], axis=1)

        cols = z_ref.shape[0]
        vrow = lax.broadcasted_iota(jnp.int32, (vocab, cols), 0)
        zrow = jnp.broadcast_to(z_ref[...][None, :], (vocab, cols))
        trow = jnp.broadcast_to(t_ref[...][None, :] + n_elements, (vocab, cols))
        mh = ((vrow == zrow) | (vrow == trow)).astype(f32)
        ht_ref[...] = jnp.dot(ct, mh, preferred_element_type=f32)

        lewt = lewt_ref[...]
        et_ref[...] = (jnp.dot(lewt[:, :rp_dim], rp_ref[...],
                               preferred_element_type=f32)
                       + jnp.dot(lewt[:, rp_dim:], ea_ref[...],
                                 preferred_element_type=f32)
                       + lebt_ref[...])

    pts = jnp.stack([period_table.astype(jnp.int32),
                     group_table.astype(jnp.int32)])
    rp_t = rel_pos.astype(jnp.float32).T
    ea_t = edge_attr.astype(jnp.float32).T
    zc = z.astype(jnp.int32)
    tc = tag.astype(jnp.int32)

    e_pad = _round_up(max(e, 1), EDGE_TILE)
    g = e_pad // EDGE_TILE
    tn = _round_up(-(-max(n, 1) // g), 128)
    n_pad = g * tn
    if n_pad != n:
        zc = jnp.pad(zc, (0, n_pad - n))
        tc = jnp.pad(tc, (0, n_pad - n))
    if e_pad != e:
        rp_t = jnp.pad(rp_t, ((0, 0), (0, e_pad - e)))
        ea_t = jnp.pad(ea_t, ((0, 0), (0, e_pad - e)))

    full = lambda i: (0, 0)
    ht, et = pl.pallas_call(
        fused_kernel,
        out_shape=(jax.ShapeDtypeStruct((hidden, n_pad), jnp.float32),
                   jax.ShapeDtypeStruct((hidden, e_pad), jnp.float32)),
        grid=(g,),
        in_specs=[
            pl.BlockSpec((tn,), lambda i: (i,)),
            pl.BlockSpec((tn,), lambda i: (i,)),
            pl.BlockSpec((2, n_elements), full),
            pl.BlockSpec((atom_dim, n_elements), full),
            pl.BlockSpec((tag_dim, n_tags), full),
            pl.BlockSpec((pg_dim, n_periods), full),
            pl.BlockSpec((pg_dim, n_groups), full),
            pl.BlockSpec((hidden, lin_w.shape[0]), full),
            pl.BlockSpec((hidden, 1), full),
            pl.BlockSpec((hidden, rp_dim + ea_dim), full),
            pl.BlockSpec((hidden, 1), full),
            pl.BlockSpec((rp_dim, EDGE_TILE), lambda i: (0, i)),
            pl.BlockSpec((ea_dim, EDGE_TILE), lambda i: (0, i)),
        ],
        out_specs=(pl.BlockSpec((hidden, tn), lambda i: (0, i)),
                   pl.BlockSpec((hidden, EDGE_TILE), lambda i: (0, i))),
        compiler_params=pltpu.CompilerParams(
            dimension_semantics=("parallel",)),
    )(zc, tc, pts,
      emb_w.astype(jnp.float32).T, tag_w.astype(jnp.float32).T,
      per_w.astype(jnp.float32).T, grp_w.astype(jnp.float32).T,
      lin_w.astype(jnp.float32).T, lin_b.astype(jnp.float32).T,
      lin_e_w.astype(jnp.float32).T, lin_e_b.astype(jnp.float32).T,
      rp_t, ea_t)

    h = ht.T if n_pad == n else ht[:, :n].T
    e_out = et.T if e_pad == e else et[:, :e].T
    return h, e_out


# speedup vs baseline: 31.9732x; 1.0736x over previous
import jax
import jax.numpy as jnp
from jax import lax
from jax.experimental import pallas as pl
from jax.experimental.pallas import tpu as pltpu

EDGE_TILE = 65536


def _round_up(x, m):
    return ((x + m - 1) // m) * m


def _dg(lhs, rhs, dims):
    return lax.dot_general(lhs, rhs, dimension_numbers=(dims, ((), ())),
                           preferred_element_type=jnp.float32)


def kernel(emb_w, tag_w, per_w, grp_w, lin_w, lin_b, lin_e_w, lin_e_b,
           period_table, group_table, z, tag, rel_pos, edge_attr):
    n = z.shape[0]
    e = rel_pos.shape[0]
    n_elements = emb_w.shape[0]
    n_tags = tag_w.shape[0]
    atom_dim = emb_w.shape[1]
    tag_dim = tag_w.shape[1]
    pg_dim = per_w.shape[1]
    n_periods = per_w.shape[0]
    n_groups = grp_w.shape[0]
    hidden = lin_w.shape[1]
    rp_dim = rel_pos.shape[1]
    ea_dim = edge_attr.shape[1]
    vocab = n_elements + n_tags

    def fused_kernel(z_ref, t_ref, pt_ref, gt_ref, emb_ref, tagw_ref, per_ref,
                     grp_ref, lw_ref, lb_ref, lew_ref, leb_ref,
                     rp_ref, ea_ref, ht_ref, et_ref):
        f32 = jnp.float32
        lw = lw_ref[...]
        o = atom_dim + tag_dim
        emb_efft = _dg(lw[:atom_dim], emb_ref[...], ((0,), (1,)))
        tag_efft = _dg(lw[atom_dim:o], tagw_ref[...], ((0,), (1,)))
        per_efft = _dg(lw[o:o + pg_dim], per_ref[...], ((0,), (1,)))
        grp_efft = _dg(lw[o + pg_dim:], grp_ref[...], ((0,), (1,)))
        pmask = (lax.broadcasted_iota(jnp.int32, (n_periods, n_elements), 0)
                 == pt_ref[...][None, :]).astype(f32)
        gmask = (lax.broadcasted_iota(jnp.int32, (n_groups, n_elements), 0)
                 == gt_ref[...][None, :]).astype(f32)
        lb_col = jnp.swapaxes(lb_ref[...], 0, 1)
        at = (emb_efft
              + jnp.dot(per_efft, pmask, preferred_element_type=f32)
              + jnp.dot(grp_efft, gmask, preferred_element_type=f32)
              + lb_col)
        ct = jnp.concatenate([at, tag_efft], axis=1)

        cols = z_ref.shape[0]
        vrow = lax.broadcasted_iota(jnp.int32, (vocab, cols), 0)
        zrow = jnp.broadcast_to(z_ref[...][None, :], (vocab, cols))
        trow = jnp.broadcast_to(t_ref[...][None, :] + n_elements, (vocab, cols))
        mh = ((vrow == zrow) | (vrow == trow)).astype(f32)
        ht_ref[...] = jnp.dot(ct, mh, preferred_element_type=f32)

        lew = lew_ref[...]
        leb_col = jnp.swapaxes(leb_ref[...], 0, 1)
        et_ref[...] = (_dg(lew[:rp_dim], rp_ref[...], ((0,), (0,)))
                       + _dg(lew[rp_dim:], ea_ref[...], ((0,), (0,)))
                       + leb_col)

    rp_t = rel_pos.astype(jnp.float32).T
    ea_t = edge_attr.astype(jnp.float32).T
    zc = z.astype(jnp.int32)
    tc = tag.astype(jnp.int32)

    e_pad = _round_up(max(e, 1), EDGE_TILE)
    g = e_pad // EDGE_TILE
    tn = _round_up(-(-max(n, 1) // g), 128)
    n_pad = g * tn
    if n_pad != n:
        zc = jnp.pad(zc, (0, n_pad - n))
        tc = jnp.pad(tc, (0, n_pad - n))
    if e_pad != e:
        rp_t = jnp.pad(rp_t, ((0, 0), (0, e_pad - e)))
        ea_t = jnp.pad(ea_t, ((0, 0), (0, e_pad - e)))

    full = lambda i: (0, 0)
    ht, et = pl.pallas_call(
        fused_kernel,
        out_shape=(jax.ShapeDtypeStruct((hidden, n_pad), jnp.float32),
                   jax.ShapeDtypeStruct((hidden, e_pad), jnp.float32)),
        grid=(g,),
        in_specs=[
            pl.BlockSpec((tn,), lambda i: (i,)),
            pl.BlockSpec((tn,), lambda i: (i,)),
            pl.BlockSpec((n_elements,), lambda i: (0,)),
            pl.BlockSpec((n_elements,), lambda i: (0,)),
            pl.BlockSpec((n_elements, atom_dim), full),
            pl.BlockSpec((n_tags, tag_dim), full),
            pl.BlockSpec((n_periods, pg_dim), full),
            pl.BlockSpec((n_groups, pg_dim), full),
            pl.BlockSpec((lin_w.shape[0], hidden), full),
            pl.BlockSpec((1, hidden), full),
            pl.BlockSpec((rp_dim + ea_dim, hidden), full),
            pl.BlockSpec((1, hidden), full),
            pl.BlockSpec((rp_dim, EDGE_TILE), lambda i: (0, i)),
            pl.BlockSpec((ea_dim, EDGE_TILE), lambda i: (0, i)),
        ],
        out_specs=(pl.BlockSpec((hidden, tn), lambda i: (0, i)),
                   pl.BlockSpec((hidden, EDGE_TILE), lambda i: (0, i))),
        compiler_params=pltpu.CompilerParams(
            dimension_semantics=("parallel",)),
    )(zc, tc, period_table.astype(jnp.int32), group_table.astype(jnp.int32),
      emb_w.astype(jnp.float32), tag_w.astype(jnp.float32),
      per_w.astype(jnp.float32), grp_w.astype(jnp.float32),
      lin_w.astype(jnp.float32), lin_b.astype(jnp.float32),
      lin_e_w.astype(jnp.float32), lin_e_b.astype(jnp.float32),
      rp_t, ea_t)

    h = ht.T if n_pad == n else ht[:, :n].T
    e_out = et.T if e_pad == e else et[:, :e].T
    return h, e_out
